# Initial kernel scaffold; baseline (speedup 1.0000x reference)
import functools

import jax
import jax.numpy as jnp
from jax import lax
from jax.experimental import pallas as pl
from jax.experimental.pallas import tpu as pltpu, tpu_sc as plsc

N_NODES = 10000
N_EDGES = 320000
EMB = 128
LANES = 16
CHUNK = 128
N_CHUNKS = N_EDGES // CHUNK
NCORES = 2
NSUB = 16
NW = NCORES * NSUB
ROWS_PER_TILE = N_NODES // NSUB
ZROWS = 125


def _spmm_partials(row, col, val, y):
    mesh = plsc.VectorSubcoreMesh(core_axis_name="c", subcore_axis_name="s")

    @functools.partial(
        pl.kernel,
        mesh=mesh,
        out_type=jax.ShapeDtypeStruct((NCORES, N_NODES, EMB), jnp.float32),
        scratch_types=[
            pltpu.VMEM((CHUNK,), jnp.int32),
            pltpu.VMEM((CHUNK,), jnp.int32),
            pltpu.VMEM((CHUNK,), jnp.float32),
            pltpu.VMEM((CHUNK, EMB), jnp.float32),
            pltpu.VMEM((ZROWS, EMB), jnp.float32),
            pltpu.VMEM_SHARED((N_NODES, EMB), jnp.float32),
            pltpu.SemaphoreType.DMA,
        ],
    )
    def k(row_h, col_h, val_h, y_h, out_h, colv, rowv, valv, rows_v, zbuf,
          acc_sh, sem):
        cid = lax.axis_index("c")
        sid = lax.axis_index("s")
        wid = cid * NSUB + sid

        def zbody(i, carry):
            r = i // (EMB // LANES)
            off = (i % (EMB // LANES)) * LANES
            zbuf[r, pl.ds(off, LANES)] = jnp.zeros((LANES,), jnp.float32)
            return carry

        lax.fori_loop(0, ZROWS * (EMB // LANES), zbody, 0)
        for t in range(ROWS_PER_TILE // ZROWS):
            pltpu.sync_copy(
                zbuf, acc_sh.at[pl.ds(sid * ROWS_PER_TILE + t * ZROWS, ZROWS)])
        plsc.subcore_barrier()

        n_mine = (N_CHUNKS - wid + NW - 1) // NW

        def chunk_body(i, carry):
            base = (wid + i * NW) * CHUNK
            pltpu.sync_copy(col_h.at[pl.ds(base, CHUNK)], colv)
            pltpu.sync_copy(row_h.at[pl.ds(base, CHUNK)], rowv)
            pltpu.sync_copy(val_h.at[pl.ds(base, CHUNK)], valv)
            pltpu.async_copy(y_h.at[colv], rows_v, sem).wait()

            def scale_body(e, c2):
                vb = jnp.full((LANES,), valv[e], jnp.float32)
                for j in range(EMB // LANES):
                    sl = pl.ds(j * LANES, LANES)
                    rows_v[e, sl] = rows_v[e, sl] * vb
                return c2

            lax.fori_loop(0, CHUNK, scale_body, 0)
            pltpu.sync_copy(rows_v, acc_sh.at[rowv], add=True)
            return carry

        lax.fori_loop(0, n_mine, chunk_body, 0)
        plsc.subcore_barrier()

        r0 = sid * ROWS_PER_TILE
        pltpu.sync_copy(acc_sh.at[pl.ds(r0, ROWS_PER_TILE)],
                        out_h.at[cid, pl.ds(r0, ROWS_PER_TILE)])

    return k(row, col, val, y)


_ROWBLK = 2000


def _gemm(x, w):
    def body(x_ref, w_ref, y_ref):
        y_ref[...] = lax.dot_general(
            x_ref[...], w_ref[...], (((1,), (1,)), ((), ())),
            preferred_element_type=jnp.float32)

    return pl.pallas_call(
        body,
        grid=(N_NODES // _ROWBLK,),
        in_specs=[
            pl.BlockSpec((_ROWBLK, EMB), lambda i: (i, 0)),
            pl.BlockSpec((EMB, EMB), lambda i: (0, 0)),
        ],
        out_specs=pl.BlockSpec((_ROWBLK, EMB), lambda i: (i, 0)),
        out_shape=jax.ShapeDtypeStruct((N_NODES, EMB), jnp.float32),
    )(x, w)


def _norm_acc_gemm(zp, acc, w):
    def body(zp_ref, acc_ref, w_ref, y_ref, accout_ref):
        z = zp_ref[0] + zp_ref[1]
        nrm = jnp.sqrt(jnp.sum(z * z, axis=-1, keepdims=True))
        xn = z / jnp.maximum(nrm, 1e-12)
        accout_ref[...] = acc_ref[...] + xn
        y_ref[...] = lax.dot_general(
            z, w_ref[...], (((1,), (1,)), ((), ())),
            preferred_element_type=jnp.float32)

    return pl.pallas_call(
        body,
        grid=(N_NODES // _ROWBLK,),
        in_specs=[
            pl.BlockSpec((NCORES, _ROWBLK, EMB), lambda i: (0, i, 0)),
            pl.BlockSpec((_ROWBLK, EMB), lambda i: (i, 0)),
            pl.BlockSpec((EMB, EMB), lambda i: (0, 0)),
        ],
        out_specs=[
            pl.BlockSpec((_ROWBLK, EMB), lambda i: (i, 0)),
            pl.BlockSpec((_ROWBLK, EMB), lambda i: (i, 0)),
        ],
        out_shape=[
            jax.ShapeDtypeStruct((N_NODES, EMB), jnp.float32),
            jax.ShapeDtypeStruct((N_NODES, EMB), jnp.float32),
        ],
    )(zp, acc, w)


def _norm_final(zp, acc):
    def body(zp_ref, acc_ref, out_ref):
        z = zp_ref[0] + zp_ref[1]
        nrm = jnp.sqrt(jnp.sum(z * z, axis=-1, keepdims=True))
        xn = z / jnp.maximum(nrm, 1e-12)
        out_ref[...] = (acc_ref[...] + xn) * 0.25

    return pl.pallas_call(
        body,
        grid=(N_NODES // _ROWBLK,),
        in_specs=[
            pl.BlockSpec((NCORES, _ROWBLK, EMB), lambda i: (0, i, 0)),
            pl.BlockSpec((_ROWBLK, EMB), lambda i: (i, 0)),
        ],
        out_specs=pl.BlockSpec((_ROWBLK, EMB), lambda i: (i, 0)),
        out_shape=jax.ShapeDtypeStruct((N_NODES, EMB), jnp.float32),
    )(zp, acc)


def kernel(adjacency_row, adjacency_col, adjacency_values, embedding, weights):
    acc = embedding
    y = _gemm(embedding, weights[0])
    zp = _spmm_partials(adjacency_row, adjacency_col, adjacency_values, y)
    y, acc = _norm_acc_gemm(zp, acc, weights[1])
    zp = _spmm_partials(adjacency_row, adjacency_col, adjacency_values, y)
    y, acc = _norm_acc_gemm(zp, acc, weights[2])
    zp = _spmm_partials(adjacency_row, adjacency_col, adjacency_values, y)
    return _norm_final(zp, acc)


# speedup vs baseline: 4.6009x; 4.6009x over previous
import functools

import jax
import jax.numpy as jnp
from jax import lax
from jax.experimental import pallas as pl
from jax.experimental.pallas import tpu as pltpu, tpu_sc as plsc

N_NODES = 10000
N_EDGES = 320000
EMB = 128
LANES = 16
CHUNK = 128
N_CHUNKS = N_EDGES // CHUNK
NCORES = 2
NSUB = 16
NW = NCORES * NSUB
ROWS_B = 624
ROWS_A = ROWS_B + 8
ZROWS = 104


def _spmm_partials(row, col, val, y):
    mesh = plsc.VectorSubcoreMesh(core_axis_name="c", subcore_axis_name="s")

    @functools.partial(
        pl.kernel,
        mesh=mesh,
        out_type=jax.ShapeDtypeStruct((NCORES, N_NODES, EMB), jnp.float32),
        scratch_types=[
            pltpu.VMEM((CHUNK,), jnp.int32),
            pltpu.VMEM((CHUNK,), jnp.int32),
            pltpu.VMEM((CHUNK,), jnp.float32),
            pltpu.VMEM((CHUNK, EMB), jnp.float32),
            pltpu.VMEM((ZROWS, EMB), jnp.float32),
            pltpu.VMEM_SHARED((N_NODES, EMB), jnp.float32),
            pltpu.SemaphoreType.DMA,
        ],
    )
    def k(row_h, col_h, val_h, y_h, out_h, colv, rowv, valv, rows_v, zbuf,
          acc_sh, sem):
        cid = lax.axis_index("c")
        sid = lax.axis_index("s")
        wid = cid * NSUB + sid

        def zbody(i, carry):
            r = i // (EMB // LANES)
            off = (i % (EMB // LANES)) * LANES
            zbuf[r, pl.ds(off, LANES)] = jnp.zeros((LANES,), jnp.float32)
            return carry

        lax.fori_loop(0, ZROWS * (EMB // LANES), zbody, 0)
        r0 = sid * ROWS_B + jnp.minimum(sid, 2) * 8
        for t in range(ROWS_B // ZROWS):
            pltpu.sync_copy(zbuf, acc_sh.at[pl.ds(r0 + t * ZROWS, ZROWS)])

        @pl.when(sid < 2)
        def _zero_tail():
            pltpu.sync_copy(zbuf.at[pl.ds(0, 8)],
                            acc_sh.at[pl.ds(r0 + ROWS_B, 8)])

        plsc.subcore_barrier()

        n_mine = (N_CHUNKS - wid + NW - 1) // NW

        def chunk_body(i, carry):
            base = (wid + i * NW) * CHUNK
            pltpu.sync_copy(col_h.at[pl.ds(base, CHUNK)], colv)
            pltpu.sync_copy(row_h.at[pl.ds(base, CHUNK)], rowv)
            pltpu.sync_copy(val_h.at[pl.ds(base, CHUNK)], valv)
            pltpu.async_copy(y_h.at[colv], rows_v, sem).wait()

            def scale_body(g, c2):
                vgrp = valv[pl.ds(g * LANES, LANES)]
                for t in range(LANES):
                    vb = jnp.full((LANES,), vgrp[t], jnp.float32)
                    e = g * LANES + t
                    for j in range(EMB // LANES):
                        sl = pl.ds(j * LANES, LANES)
                        rows_v[e, sl] = rows_v[e, sl] * vb
                return c2

            lax.fori_loop(0, CHUNK // LANES, scale_body, 0)
            pltpu.sync_copy(rows_v, acc_sh.at[rowv], add=True)
            return carry

        lax.fori_loop(0, n_mine, chunk_body, 0)
        plsc.subcore_barrier()

        @pl.when(sid < 2)
        def _pub_a():
            pltpu.sync_copy(acc_sh.at[pl.ds(r0, ROWS_A)],
                            out_h.at[cid, pl.ds(r0, ROWS_A)])

        @pl.when(sid >= 2)
        def _pub_b():
            pltpu.sync_copy(acc_sh.at[pl.ds(r0, ROWS_B)],
                            out_h.at[cid, pl.ds(r0, ROWS_B)])

    return k(row, col, val, y)


_ROWBLK = 2000


def _gemm(x, w):
    def body(x_ref, w_ref, y_ref):
        y_ref[...] = lax.dot_general(
            x_ref[...], w_ref[...], (((1,), (1,)), ((), ())),
            preferred_element_type=jnp.float32)

    return pl.pallas_call(
        body,
        grid=(N_NODES // _ROWBLK,),
        in_specs=[
            pl.BlockSpec((_ROWBLK, EMB), lambda i: (i, 0)),
            pl.BlockSpec((EMB, EMB), lambda i: (0, 0)),
        ],
        out_specs=pl.BlockSpec((_ROWBLK, EMB), lambda i: (i, 0)),
        out_shape=jax.ShapeDtypeStruct((N_NODES, EMB), jnp.float32),
    )(x, w)


def _norm_acc_gemm(zp, acc, w):
    def body(zp_ref, acc_ref, w_ref, y_ref, accout_ref):
        z = zp_ref[0] + zp_ref[1]
        nrm = jnp.sqrt(jnp.sum(z * z, axis=-1, keepdims=True))
        xn = z / jnp.maximum(nrm, 1e-12)
        accout_ref[...] = acc_ref[...] + xn
        y_ref[...] = lax.dot_general(
            z, w_ref[...], (((1,), (1,)), ((), ())),
            preferred_element_type=jnp.float32)

    return pl.pallas_call(
        body,
        grid=(N_NODES // _ROWBLK,),
        in_specs=[
            pl.BlockSpec((NCORES, _ROWBLK, EMB), lambda i: (0, i, 0)),
            pl.BlockSpec((_ROWBLK, EMB), lambda i: (i, 0)),
            pl.BlockSpec((EMB, EMB), lambda i: (0, 0)),
        ],
        out_specs=[
            pl.BlockSpec((_ROWBLK, EMB), lambda i: (i, 0)),
            pl.BlockSpec((_ROWBLK, EMB), lambda i: (i, 0)),
        ],
        out_shape=[
            jax.ShapeDtypeStruct((N_NODES, EMB), jnp.float32),
            jax.ShapeDtypeStruct((N_NODES, EMB), jnp.float32),
        ],
    )(zp, acc, w)


def _norm_final(zp, acc):
    def body(zp_ref, acc_ref, out_ref):
        z = zp_ref[0] + zp_ref[1]
        nrm = jnp.sqrt(jnp.sum(z * z, axis=-1, keepdims=True))
        xn = z / jnp.maximum(nrm, 1e-12)
        out_ref[...] = (acc_ref[...] + xn) * 0.25

    return pl.pallas_call(
        body,
        grid=(N_NODES // _ROWBLK,),
        in_specs=[
            pl.BlockSpec((NCORES, _ROWBLK, EMB), lambda i: (0, i, 0)),
            pl.BlockSpec((_ROWBLK, EMB), lambda i: (i, 0)),
        ],
        out_specs=pl.BlockSpec((_ROWBLK, EMB), lambda i: (i, 0)),
        out_shape=jax.ShapeDtypeStruct((N_NODES, EMB), jnp.float32),
    )(zp, acc)


def kernel(adjacency_row, adjacency_col, adjacency_values, embedding, weights):
    acc = embedding
    y = _gemm(embedding, weights[0])
    zp = _spmm_partials(adjacency_row, adjacency_col, adjacency_values, y)
    y, acc = _norm_acc_gemm(zp, acc, weights[1])
    zp = _spmm_partials(adjacency_row, adjacency_col, adjacency_values, y)
    y, acc = _norm_acc_gemm(zp, acc, weights[2])
    zp = _spmm_partials(adjacency_row, adjacency_col, adjacency_values, y)
    return _norm_final(zp, acc)
